# baseline (speedup 1.0000x reference)
import jax
import jax.numpy as jnp
from jax.experimental import pallas as pl
from jax.experimental.pallas import tpu as pltpu

_B, _N, _D_IN = 8, 1024, 768
_K, _E = 8192, 64
_T = 256
_NT = (_B * _N) // _T


def _rpq_body(x_ref, rp_ref, cb_ref, out_ref, cn_ref, c2_ref):
    i = pl.program_id(0)

    @pl.when(i == 0)
    def _():
        cb = cb_ref[...]
        cnorm = jnp.sqrt(jnp.sum(cb * cb, axis=1, keepdims=True))
        cn = cb / jnp.maximum(cnorm, 1e-12)
        cn_ref[...] = cn
        c2_ref[...] = jnp.sum(cn * cn, axis=1, keepdims=True)

    proj = jax.lax.dot_general(
        x_ref[...], rp_ref[...], (((1,), (0,)), ((), ())),
        preferred_element_type=jnp.float32)
    xnorm = jnp.sqrt(jnp.sum(proj * proj, axis=1, keepdims=True))
    xn = proj / jnp.maximum(xnorm, 1e-12)
    x2 = jnp.sum(xn * xn, axis=1, keepdims=True).reshape(1, -1)

    scores = jax.lax.dot_general(
        cn_ref[...], xn, (((1,), (1,)), ((), ())),
        preferred_element_type=jnp.float32)
    d2 = c2_ref[...] + x2 - 2.0 * scores

    m = jnp.min(d2, axis=0, keepdims=True)
    sm = jnp.sqrt(jnp.maximum(m, 0.0))
    h0b = jax.lax.bitcast_convert_type(sm * sm, jnp.int32)
    h = m
    for j in range(-2, 4):
        xj = jax.lax.bitcast_convert_type(h0b + j, jnp.float32)
        ok = jnp.sqrt(jnp.maximum(xj, 0.0)) == sm
        h = jnp.where(ok, jnp.maximum(h, xj), h)
    ks = jax.lax.broadcasted_iota(jnp.int32, d2.shape, 0)
    out_ref[0, 0, :] = jnp.min(jnp.where(d2 <= h, ks, _K), axis=0)


def kernel(x, random_projection, codebook):
    b, n, _ = x.shape
    x_flat = x.reshape(b * n, _D_IN)
    out = pl.pallas_call(
        _rpq_body,
        grid=(_NT,),
        in_specs=[
            pl.BlockSpec((_T, _D_IN), lambda i: (i, 0)),
            pl.BlockSpec((_D_IN, _E), lambda i: (0, 0)),
            pl.BlockSpec((_K, _E), lambda i: (0, 0)),
        ],
        out_specs=pl.BlockSpec((1, 1, _T), lambda i: (i, 0, 0)),
        out_shape=jax.ShapeDtypeStruct((_NT, 1, _T), jnp.int32),
        scratch_shapes=[
            pltpu.VMEM((_K, _E), jnp.float32),
            pltpu.VMEM((_K, 1), jnp.float32),
        ],
        compiler_params=pltpu.CompilerParams(
            dimension_semantics=("arbitrary",)),
    )(x_flat, random_projection, codebook)
    return out.reshape(b, n)


# speedup vs baseline: 1.1637x; 1.1637x over previous
import jax
import jax.numpy as jnp
from jax.experimental import pallas as pl
from jax.experimental.pallas import tpu as pltpu

_B, _N, _D_IN = 8, 1024, 768
_K, _E = 8192, 64
_T = 512
_NT = (_B * _N) // _T
_NC = 8
_CK = _K // _NC


def _rpq_body(x_ref, rp_ref, cb_ref, out_ref, cn_ref, c2_ref, d2_ref, h_ref):
    i = pl.program_id(0)
    p = jax.lax.rem(i, 2)
    q = jax.lax.rem(i + 1, 2)

    @pl.when(i == 0)
    def _():
        cb = cb_ref[...]
        cnorm = jnp.sqrt(jnp.sum(cb * cb, axis=1, keepdims=True))
        cn = cb / jnp.maximum(cnorm, 1e-12)
        cn_ref[...] = cn
        c2_ref[...] = jnp.sum(cn * cn, axis=1, keepdims=True)

    proj = jax.lax.dot_general(
        x_ref[...], rp_ref[...], (((1,), (0,)), ((), ())),
        preferred_element_type=jnp.float32)
    xnorm = jnp.sqrt(jnp.sum(proj * proj, axis=1, keepdims=True))
    xn = proj / jnp.maximum(xnorm, 1e-12)
    x2 = jnp.sum(xn * xn, axis=1, keepdims=True).reshape(1, -1)

    scores = jax.lax.dot_general(
        cn_ref[...], xn, (((1,), (1,)), ((), ())),
        preferred_element_type=jnp.float32)
    d2 = c2_ref[...] + x2 - 2.0 * scores
    d2_ref[p] = d2

    m = jnp.min(d2, axis=0, keepdims=True)
    sm = jnp.sqrt(jnp.maximum(m, 0.0))
    h0b = jax.lax.bitcast_convert_type(sm * sm, jnp.int32)
    h = m
    for j in range(-2, 4):
        xj = jax.lax.bitcast_convert_type(h0b + j, jnp.float32)
        ok = jnp.sqrt(jnp.maximum(xj, 0.0)) == sm
        h = jnp.where(ok, jnp.maximum(h, xj), h)
    h_ref[p] = h

    hq = h_ref[q]
    idx = None
    for c in range(_NC):
        d2c = d2_ref[q, pl.ds(c * _CK, _CK), :]
        ks = jax.lax.broadcasted_iota(jnp.int32, (_CK, _T), 0) + c * _CK
        ic = jnp.min(jnp.where(d2c <= hq, ks, _K), axis=0)
        idx = ic if idx is None else jnp.minimum(idx, ic)
    out_ref[0, 0, :] = idx


def kernel(x, random_projection, codebook):
    b, n, _ = x.shape
    x_flat = x.reshape(b * n, _D_IN)
    out = pl.pallas_call(
        _rpq_body,
        grid=(_NT + 1,),
        in_specs=[
            pl.BlockSpec((_T, _D_IN), lambda i: (jnp.minimum(i, _NT - 1), 0)),
            pl.BlockSpec((_D_IN, _E), lambda i: (0, 0)),
            pl.BlockSpec((_K, _E), lambda i: (0, 0)),
        ],
        out_specs=pl.BlockSpec(
            (1, 1, _T), lambda i: (jnp.maximum(i - 1, 0), 0, 0)),
        out_shape=jax.ShapeDtypeStruct((_NT, 1, _T), jnp.int32),
        scratch_shapes=[
            pltpu.VMEM((_K, _E), jnp.float32),
            pltpu.VMEM((_K, 1), jnp.float32),
            pltpu.VMEM((2, _K, _T), jnp.float32),
            pltpu.VMEM((2, 1, _T), jnp.float32),
        ],
        compiler_params=pltpu.CompilerParams(
            dimension_semantics=("arbitrary",)),
    )(x_flat, random_projection, codebook)
    return out.reshape(b, n)


# speedup vs baseline: 1.3320x; 1.1446x over previous
import jax
import jax.numpy as jnp
from jax.experimental import pallas as pl
from jax.experimental.pallas import tpu as pltpu

_B, _N, _D_IN = 8, 1024, 768
_K, _E = 8192, 64
_T = 512
_NT = (_B * _N) // _T
_NC = 8
_CK = _K // _NC


def _rpq_body(x_ref, rp_ref, cb_ref, out_ref, cn_ref, c2_ref, d2_ref, h_ref,
              ks_ref):
    i = pl.program_id(0)
    p = jax.lax.rem(i, 2)
    q = jax.lax.rem(i + 1, 2)

    @pl.when(i == 0)
    def _():
        cb = cb_ref[...]
        cnorm = jnp.sqrt(jnp.sum(cb * cb, axis=1, keepdims=True))
        cn = cb / jnp.maximum(cnorm, 1e-12)
        cn_ref[...] = cn
        c2_ref[...] = jnp.sum(cn * cn, axis=1, keepdims=True)
        ks_ref[...] = jax.lax.broadcasted_iota(
            jnp.int32, (_CK, _T), 0).astype(jnp.float32)

    proj = jax.lax.dot_general(
        x_ref[...], rp_ref[...], (((1,), (0,)), ((), ())),
        preferred_element_type=jnp.float32)
    xnorm = jnp.sqrt(jnp.sum(proj * proj, axis=1, keepdims=True))
    xn = proj / jnp.maximum(xnorm, 1e-12)
    x2 = jnp.sum(xn * xn, axis=1, keepdims=True).reshape(1, -1)

    scores = jax.lax.dot_general(
        cn_ref[...], xn, (((1,), (1,)), ((), ())),
        preferred_element_type=jnp.float32)
    d2 = c2_ref[...] + x2 - 2.0 * scores
    d2_ref[p] = d2

    m = jnp.min(d2, axis=0, keepdims=True)
    sm = jnp.sqrt(jnp.maximum(m, 0.0))
    h0b = jax.lax.bitcast_convert_type(sm * sm, jnp.int32)
    h = m
    for j in range(-2, 4):
        xj = jax.lax.bitcast_convert_type(h0b + j, jnp.float32)
        ok = jnp.sqrt(jnp.maximum(xj, 0.0)) == sm
        h = jnp.where(ok, jnp.maximum(h, xj), h)
    h_ref[p] = h

    hq = h_ref[q]
    ksf = ks_ref[...]
    idx = None
    for c in range(_NC):
        d2c = d2_ref[q, pl.ds(c * _CK, _CK), :]
        ic = (jnp.min(jnp.where(d2c <= hq, ksf, jnp.float32(_K)), axis=0)
              + jnp.float32(c * _CK))
        idx = ic if idx is None else jnp.minimum(idx, ic)
    out_ref[0, 0, :] = idx.astype(jnp.int32)


def kernel(x, random_projection, codebook):
    b, n, _ = x.shape
    x_flat = x.reshape(b * n, _D_IN)
    out = pl.pallas_call(
        _rpq_body,
        grid=(_NT + 1,),
        in_specs=[
            pl.BlockSpec((_T, _D_IN), lambda i: (jnp.minimum(i, _NT - 1), 0)),
            pl.BlockSpec((_D_IN, _E), lambda i: (0, 0)),
            pl.BlockSpec((_K, _E), lambda i: (0, 0)),
        ],
        out_specs=pl.BlockSpec(
            (1, 1, _T), lambda i: (jnp.maximum(i - 1, 0), 0, 0)),
        out_shape=jax.ShapeDtypeStruct((_NT, 1, _T), jnp.int32),
        scratch_shapes=[
            pltpu.VMEM((_K, _E), jnp.float32),
            pltpu.VMEM((_K, 1), jnp.float32),
            pltpu.VMEM((2, _K, _T), jnp.float32),
            pltpu.VMEM((2, 1, _T), jnp.float32),
            pltpu.VMEM((_CK, _T), jnp.float32),
        ],
        compiler_params=pltpu.CompilerParams(
            dimension_semantics=("arbitrary",),
            vmem_limit_bytes=62 * 1024 * 1024),
    )(x_flat, random_projection, codebook)
    return out.reshape(b, n)


# speedup vs baseline: 1.4220x; 1.0676x over previous
import jax
import jax.numpy as jnp
from jax.experimental import pallas as pl
from jax.experimental.pallas import tpu as pltpu

_B, _N, _D_IN = 8, 1024, 768
_K, _E = 8192, 64
_T = 512
_NT = (_B * _N) // _T
_NC = 8
_CK = _K // _NC


def _rpq_body(x_ref, rp_ref, cb_ref, out_ref, cn_ref, c2_ref, d2_ref, h_ref,
              ks_ref):
    i = pl.program_id(0)
    p = jax.lax.rem(i, 2)
    q = jax.lax.rem(i + 1, 2)

    @pl.when(i == 0)
    def _():
        cb = cb_ref[...]
        cnorm = jnp.sqrt(jnp.sum(cb * cb, axis=1, keepdims=True))
        cn = cb / jnp.maximum(cnorm, 1e-12)
        cn_ref[...] = -2.0 * cn
        c2_ref[...] = jnp.sum(cn * cn, axis=1, keepdims=True)
        ks_ref[...] = jax.lax.broadcasted_iota(
            jnp.int32, (_CK, _T), 0).astype(jnp.float32)

    proj = jax.lax.dot_general(
        x_ref[...], rp_ref[...], (((1,), (0,)), ((), ())),
        preferred_element_type=jnp.float32)
    xnorm = jnp.sqrt(jnp.sum(proj * proj, axis=1, keepdims=True))
    xn = proj / jnp.maximum(xnorm, 1e-12)
    x2 = jnp.sum(xn * xn, axis=1, keepdims=True).reshape(1, -1)

    scores = jax.lax.dot_general(
        cn_ref[...], xn, (((1,), (1,)), ((), ())),
        preferred_element_type=jnp.float32)
    d2 = (c2_ref[...] + x2) + scores
    d2_ref[p] = d2

    m = jnp.min(d2, axis=0, keepdims=True)
    sm = jnp.sqrt(jnp.maximum(m, 0.0))
    h0b = jax.lax.bitcast_convert_type(sm * sm, jnp.int32)
    h = m
    for j in range(-2, 4):
        xj = jax.lax.bitcast_convert_type(h0b + j, jnp.float32)
        ok = jnp.sqrt(jnp.maximum(xj, 0.0)) == sm
        h = jnp.where(ok, jnp.maximum(h, xj), h)
    h_ref[p] = h

    hq = h_ref[q]
    ksf = ks_ref[...]
    idx = None
    for c in range(_NC):
        d2c = d2_ref[q, pl.ds(c * _CK, _CK), :]
        ic = (jnp.min(jnp.where(d2c <= hq, ksf, jnp.float32(_K)), axis=0)
              + jnp.float32(c * _CK))
        idx = ic if idx is None else jnp.minimum(idx, ic)
    out_ref[0, 0, :] = idx.astype(jnp.int32)


def kernel(x, random_projection, codebook):
    b, n, _ = x.shape
    x_flat = x.reshape(b * n, _D_IN)
    out = pl.pallas_call(
        _rpq_body,
        grid=(_NT + 1,),
        in_specs=[
            pl.BlockSpec((_T, _D_IN), lambda i: (jnp.minimum(i, _NT - 1), 0)),
            pl.BlockSpec((_D_IN, _E), lambda i: (0, 0)),
            pl.BlockSpec((_K, _E), lambda i: (0, 0)),
        ],
        out_specs=pl.BlockSpec(
            (1, 1, _T), lambda i: (jnp.maximum(i - 1, 0), 0, 0)),
        out_shape=jax.ShapeDtypeStruct((_NT, 1, _T), jnp.int32),
        scratch_shapes=[
            pltpu.VMEM((_K, _E), jnp.float32),
            pltpu.VMEM((_K, 1), jnp.float32),
            pltpu.VMEM((2, _K, _T), jnp.float32),
            pltpu.VMEM((2, 1, _T), jnp.float32),
            pltpu.VMEM((_CK, _T), jnp.float32),
        ],
        compiler_params=pltpu.CompilerParams(
            dimension_semantics=("arbitrary",),
            vmem_limit_bytes=62 * 1024 * 1024),
    )(x_flat, random_projection, codebook)
    return out.reshape(b, n)
